# baseline (speedup 1.0000x reference)
import functools

import jax
import jax.numpy as jnp
from jax import lax
from jax.experimental import pallas as pl
from jax.experimental.pallas import tpu as pltpu
from jax.experimental.pallas import tpu_sc as plsc

N, C, H, W = 64, 384, 28, 28
HW = H * W
LANES = 16
C_CHUNKS = C // LANES
ROWS_PER_DMA = 64
CHUNKS_PER_BATCH = C // ROWS_PER_DMA
BATCHES_PER_WORKER = 2
CHUNKS_PER_WORKER = BATCHES_PER_WORKER * CHUNKS_PER_BATCH


def _body(x_hbm, mask_hbm, out_hbm, mask_v, sel_v, rowidx_v, buf_v,
          gsem0, gsem1, ssem0, ssem1):
    info = plsc.get_sparse_core_info()
    wid = lax.axis_index("s") * info.num_cores + lax.axis_index("c")

    pltpu.sync_copy(mask_hbm, mask_v)

    zero16 = jnp.zeros((LANES,), jnp.int32)
    zero16f = jnp.zeros((LANES,), jnp.float32)
    one16 = jnp.ones((LANES,), jnp.int32)
    iota16 = lax.iota(jnp.int32, LANES)
    for t in range(C_CHUNKS):
        sel_v[pl.ds(t * LANES, LANES)] = zero16
    offset = jnp.int32(0)
    for t in range(C_CHUNKS):
        xv = mask_v[pl.ds(t * LANES, LANES)]
        m = xv != zero16f
        mi = m.astype(jnp.int32)
        pos = plsc.cumsum(mi)
        ids = iota16 + jnp.full((LANES,), t * LANES, jnp.int32)
        offv = lax.broadcast_in_dim(offset, (LANES,), ())
        plsc.store_scatter(sel_v, [pos + offv - one16], ids, mask=m)
        offset = offset + jnp.sum(mi)

    n0 = wid * BATCHES_PER_WORKER
    for b in range(BATCHES_PER_WORKER):
        base_row = lax.broadcast_in_dim((n0 + b) * C, (LANES,), ())
        for t in range(C_CHUNKS):
            c0 = t * LANES
            g = b * CHUNKS_PER_BATCH + c0 // ROWS_PER_DMA
            off = c0 % ROWS_PER_DMA
            rowidx_v[g, pl.ds(off, LANES)] = sel_v[pl.ds(c0, LANES)] + base_row

    gsems = (gsem0, gsem1)
    ssems = (ssem0, ssem1)

    def start_gather(g):
        slot = g % 2
        return pltpu.async_copy(
            x_hbm.at[rowidx_v.at[g]], buf_v.at[slot], gsems[slot])

    def start_scatter(g):
        slot = g % 2
        b, ch = g // CHUNKS_PER_BATCH, g % CHUNKS_PER_BATCH
        row0 = (n0 + b) * C + ch * ROWS_PER_DMA
        return pltpu.async_copy(
            buf_v.at[slot], out_hbm.at[pl.ds(row0, ROWS_PER_DMA)], ssems[slot])

    gathers = [start_gather(0)]
    scatters = []
    for g in range(CHUNKS_PER_WORKER):
        if g + 1 < CHUNKS_PER_WORKER:
            if g >= 1:
                scatters[g - 1].wait()
            gathers.append(start_gather(g + 1))
        gathers[g].wait()
        scatters.append(start_scatter(g))
    scatters[-2].wait()
    scatters[-1].wait()


def kernel(input_tensor, indexes):
    x = input_tensor.reshape(N * C, HW)
    mesh = plsc.VectorSubcoreMesh(core_axis_name="c", subcore_axis_name="s")
    run = functools.partial(
        pl.kernel,
        mesh=mesh,
        compiler_params=pltpu.CompilerParams(
            use_tc_tiling_on_sc=False, needs_layout_passes=False),
        out_type=jax.ShapeDtypeStruct((N * C, HW), jnp.float32),
        scratch_types=[
            pltpu.VMEM((C,), jnp.float32),
            pltpu.VMEM((C,), jnp.int32),
            pltpu.VMEM((CHUNKS_PER_WORKER, ROWS_PER_DMA), jnp.int32),
            pltpu.VMEM((2, ROWS_PER_DMA, HW), jnp.float32),
            pltpu.SemaphoreType.DMA,
            pltpu.SemaphoreType.DMA,
            pltpu.SemaphoreType.DMA,
            pltpu.SemaphoreType.DMA,
        ],
    )(_body)
    out = run(x, indexes)
    return out.reshape(N, C, H, W)


# speedup vs baseline: 1.2924x; 1.2924x over previous
import functools

import jax
import jax.numpy as jnp
from jax import lax
from jax.experimental import pallas as pl
from jax.experimental.pallas import tpu as pltpu
from jax.experimental.pallas import tpu_sc as plsc

N, C, H, W = 64, 384, 28, 28
LANES = 16
C_CHUNKS = C // LANES
PLANES_PER_DMA = 8
CHUNKS_PER_BATCH = C // PLANES_PER_DMA
BATCHES_PER_WORKER = 2
CHUNKS_PER_WORKER = BATCHES_PER_WORKER * CHUNKS_PER_BATCH
PAIRS = CHUNKS_PER_WORKER // 2


def _body(x_hbm, mask_hbm, out_hbm, mask_v, sel_v, rowidx_v, buf_v,
          gsem0, gsem1, ssem0, ssem1):
    info = plsc.get_sparse_core_info()
    wid = lax.axis_index("s") * info.num_cores + lax.axis_index("c")

    pltpu.sync_copy(mask_hbm, mask_v)

    zero16 = jnp.zeros((LANES,), jnp.int32)
    zero16f = jnp.zeros((LANES,), jnp.float32)
    one16 = jnp.ones((LANES,), jnp.int32)
    iota16 = lax.iota(jnp.int32, LANES)
    for t in range(C_CHUNKS):
        sel_v[pl.ds(t * LANES, LANES)] = zero16
    offset = jnp.int32(0)
    for t in range(C_CHUNKS):
        xv = mask_v[pl.ds(t * LANES, LANES)]
        m = xv != zero16f
        mi = m.astype(jnp.int32)
        pos = plsc.cumsum(mi)
        ids = iota16 + jnp.full((LANES,), t * LANES, jnp.int32)
        offv = lax.broadcast_in_dim(offset, (LANES,), ())
        plsc.store_scatter(sel_v, [pos + offv - one16], ids, mask=m)
        offset = offset + jnp.sum(mi)

    n0 = wid * BATCHES_PER_WORKER
    for b in range(BATCHES_PER_WORKER):
        base_row = lax.broadcast_in_dim((n0 + b) * C, (LANES,), ())
        for t in range(C_CHUNKS):
            c0 = t * LANES
            rowidx_v[pl.ds(b * C + c0, LANES)] = (
                sel_v[pl.ds(c0, LANES)] + base_row)

    gsems = (gsem0, gsem1)
    ssems = (ssem0, ssem1)

    def out_row0(g):
        b = g // CHUNKS_PER_BATCH
        ch = lax.rem(g, jnp.int32(CHUNKS_PER_BATCH))
        return (n0 + b) * C + ch * PLANES_PER_DMA

    def start_gather(g, slot):
        vec = rowidx_v[pl.ds(g * PLANES_PER_DMA, LANES)]
        first = vec[0]
        firstv = lax.broadcast_in_dim(first, (LANES,), ())
        eq = vec == firstv + iota16
        in_chunk = iota16 < jnp.full((LANES,), PLANES_PER_DMA, jnp.int32)
        contig = jnp.all(jnp.logical_or(eq, jnp.logical_not(in_chunk)))

        @pl.when(contig)
        def _():
            pltpu.async_copy(
                x_hbm.at[pl.ds(first, PLANES_PER_DMA)], buf_v.at[slot],
                gsems[slot])

        @pl.when(jnp.logical_not(contig))
        def _():
            for j in range(PLANES_PER_DMA):
                pltpu.async_copy(
                    x_hbm.at[vec[j]], buf_v.at[slot, j],
                    gsems[slot])

    def wait_gather(slot):
        pltpu.make_async_copy(
            x_hbm.at[pl.ds(0, PLANES_PER_DMA)], buf_v.at[slot],
            gsems[slot]).wait()

    def start_scatter(g, slot):
        pltpu.async_copy(
            buf_v.at[slot], out_hbm.at[pl.ds(out_row0(g), PLANES_PER_DMA)],
            ssems[slot])

    def wait_scatter(g, slot):
        pltpu.make_async_copy(
            buf_v.at[slot], out_hbm.at[pl.ds(out_row0(g), PLANES_PER_DMA)],
            ssems[slot]).wait()

    def pair(i, carry):
        ga = 2 * i
        gb = ga + 1

        @pl.when(i > 0)
        def _():
            wait_scatter(ga - 2, 0)
        start_gather(ga, 0)

        @pl.when(i > 0)
        def _():
            wait_scatter(gb - 2, 1)
        start_gather(gb, 1)

        wait_gather(0)
        start_scatter(ga, 0)
        wait_gather(1)
        start_scatter(gb, 1)
        return carry

    lax.fori_loop(0, PAIRS, pair, jnp.int32(0))
    wait_scatter(CHUNKS_PER_WORKER - 2, 0)
    wait_scatter(CHUNKS_PER_WORKER - 1, 1)


def kernel(input_tensor, indexes):
    x = input_tensor.reshape(N * C, H, W)
    mesh = plsc.VectorSubcoreMesh(core_axis_name="c", subcore_axis_name="s")
    run = functools.partial(
        pl.kernel,
        mesh=mesh,
        compiler_params=pltpu.CompilerParams(
            use_tc_tiling_on_sc=True, needs_layout_passes=False),
        out_type=jax.ShapeDtypeStruct((N * C, H, W), jnp.float32),
        scratch_types=[
            pltpu.VMEM((C,), jnp.float32),
            pltpu.VMEM((C,), jnp.int32),
            pltpu.VMEM((BATCHES_PER_WORKER * C + LANES,), jnp.int32),
            pltpu.VMEM((2, PLANES_PER_DMA, H, W), jnp.float32),
            pltpu.SemaphoreType.DMA,
            pltpu.SemaphoreType.DMA,
            pltpu.SemaphoreType.DMA,
            pltpu.SemaphoreType.DMA,
        ],
    )(_body)
    out = run(x, indexes)
    return out.reshape(N, C, H, W)
